# baseline (speedup 1.0000x reference)
import functools

import jax
import jax.numpy as jnp
from jax import lax
from jax.experimental import pallas as pl
from jax.experimental.pallas import tpu as pltpu
from jax.experimental.pallas import tpu_sc as plsc

NC, NS, L = 2, 16, 16
NW = NC * NS
CHUNK = 400
REC = 144

_SC_PARAMS = pltpu.CompilerParams(use_tc_tiling_on_sc=False)


def _sc_mesh():
    return plsc.VectorSubcoreMesh(
        core_axis_name="c", subcore_axis_name="s", num_cores=NC, num_subcores=NS
    )


def _make_gather(N, E):
    epw = E // NW
    n_chunks = epw // CHUNK

    @functools.partial(
        pl.kernel,
        out_type=[
            jax.ShapeDtypeStruct((E, REC), jnp.float32),
            jax.ShapeDtypeStruct((E, REC), jnp.float32),
        ],
        mesh=_sc_mesh(),
        compiler_params=_SC_PARAMS,
        scratch_types=[
            pltpu.VMEM((CHUNK,), jnp.int32),
            pltpu.VMEM((CHUNK,), jnp.int32),
            pltpu.VMEM((CHUNK, REC), jnp.float32),
            pltpu.VMEM((CHUNK, REC), jnp.float32),
            pltpu.SemaphoreType.DMA,
            pltpu.SemaphoreType.DMA,
        ],
    )
    def gather_k(hx_hbm, row_hbm, col_hbm, hs_hbm, hd_hbm,
                 ir_v, ic_v, gs, gd, sem1, sem2):
        wid = lax.axis_index("s") * NC + lax.axis_index("c")
        base_w = wid * epw

        def chunk_body(ci, _):
            base = base_w + ci * CHUNK
            pltpu.sync_copy(row_hbm.at[pl.ds(base, CHUNK)], ir_v)
            pltpu.sync_copy(col_hbm.at[pl.ds(base, CHUNK)], ic_v)
            cp1 = pltpu.make_async_copy(hx_hbm.at[ir_v], gs, sem1)
            cp2 = pltpu.make_async_copy(hx_hbm.at[ic_v], gd, sem2)
            cp1.start()
            cp2.start()
            cp1.wait()
            pltpu.sync_copy(gs, hs_hbm.at[pl.ds(base, CHUNK)])
            cp2.wait()
            pltpu.sync_copy(gd, hd_hbm.at[pl.ds(base, CHUNK)])
            return _

        lax.fori_loop(0, n_chunks, chunk_body, None)

    return gather_k


SCHUNK = 200


def _make_scatter(N, E, D):
    epw = E // NW
    n_chunks = epw // SCHUNK
    npt = N // NS

    @functools.partial(
        pl.kernel,
        out_type=jax.ShapeDtypeStruct((NC * N, D), jnp.float32),
        mesh=_sc_mesh(),
        compiler_params=_SC_PARAMS,
        scratch_types=[
            pltpu.VMEM((SCHUNK, D), jnp.float32),
            pltpu.VMEM((SCHUNK,), jnp.int32),
            pltpu.VMEM_SHARED((N, D), jnp.float32),
        ],
    )
    def scatter_k(mij_hbm, row_hbm, out_hbm, mbuf, idxv, agg_sh):
        cid = lax.axis_index("c")
        sid = lax.axis_index("s")
        wid = sid * NC + cid
        base_w = wid * epw

        def zrow(rr, _):
            def zcol(cc, _2):
                mbuf[rr, pl.ds(cc * L, L)] = jnp.zeros((L,), jnp.float32)
                return _2
            return lax.fori_loop(0, D // L, zcol, _)

        lax.fori_loop(0, SCHUNK, zrow, None)
        done = 0
        while done < npt:
            step = min(SCHUNK, npt - done)
            pltpu.sync_copy(mbuf.at[pl.ds(0, step)],
                            agg_sh.at[pl.ds(sid * npt + done, step)])
            done += step
        plsc.subcore_barrier()

        def chunk_body(ci, _):
            base = base_w + ci * SCHUNK
            pltpu.sync_copy(row_hbm.at[pl.ds(base, SCHUNK)], idxv)
            pltpu.sync_copy(mij_hbm.at[pl.ds(base, SCHUNK)], mbuf)
            pltpu.sync_copy(mbuf, agg_sh.at[idxv], add=True)
            return _

        lax.fori_loop(0, n_chunks, chunk_body, None)
        plsc.subcore_barrier()
        pltpu.sync_copy(agg_sh.at[pl.ds(sid * npt, npt)],
                        out_hbm.at[pl.ds(cid * N + sid * npt, npt)])

    return scatter_k


def _edge_body(hxs_ref, hxd_ref, bond_ref,
               w1a_ref, w1b_ref, w1c_ref, w1d_ref, b1_ref, w2_ref, b2_ref,
               bw1a_ref, bw1b_ref, bb1_ref, bw2_ref, bb2_ref,
               off_ref, cf_ref, mij_ref, bout_ref):
    f32 = jnp.float32
    bf16 = jnp.bfloat16
    D = w1a_ref.shape[0]
    hs = hxs_ref[:, :D]
    hd = hxd_ref[:, :D]
    xs = hxs_ref[:, D:D + 3]
    xd = hxd_ref[:, D:D + 3]
    bond = bond_ref[...]
    dx = xs - xd
    r2 = jnp.sum(dx * dx, axis=1, keepdims=True)
    r = jnp.sqrt(r2 + 1e-8)
    rbf = jnp.exp(cf_ref[0, 0] * (r - off_ref[...]) ** 2)

    d1 = 2 * D + bond.shape[1] + rbf.shape[1]
    colsum1 = (jnp.sum(w1a_ref[...], 0) + jnp.sum(w1b_ref[...], 0)
               + jnp.sum(w1c_ref[...], 0) + jnp.sum(w1d_ref[...], 0))[None, :]
    s = (jnp.sum(hs, 1, keepdims=True) + jnp.sum(hd, 1, keepdims=True)
         + jnp.sum(bond, 1, keepdims=True) + jnp.sum(rbf, 1, keepdims=True))
    q = (jnp.sum(hs * hs, 1, keepdims=True) + jnp.sum(hd * hd, 1, keepdims=True)
         + jnp.sum(bond * bond, 1, keepdims=True)
         + jnp.sum(rbf * rbf, 1, keepdims=True))
    mean = s / d1
    inv = lax.rsqrt(q / d1 - mean * mean + 1e-5)
    msum = (jnp.dot(hs.astype(bf16), w1a_ref[...].astype(bf16),
                    preferred_element_type=f32)
            + jnp.dot(hd.astype(bf16), w1b_ref[...].astype(bf16),
                      preferred_element_type=f32)
            + jnp.dot(bond.astype(bf16), w1c_ref[...].astype(bf16),
                      preferred_element_type=f32)
            + jnp.dot(rbf.astype(bf16), w1d_ref[...].astype(bf16),
                      preferred_element_type=f32))
    u1 = (msum - mean * colsum1) * inv + b1_ref[...]
    t1 = u1 * jax.nn.sigmoid(u1)
    m1 = jnp.mean(t1, -1, keepdims=True)
    c1 = t1 - m1
    t1n = c1 * lax.rsqrt(jnp.mean(c1 * c1, -1, keepdims=True) + 1e-5)
    u2 = (jnp.dot(t1n.astype(bf16), w2_ref[...].astype(bf16),
                  preferred_element_type=f32) + b2_ref[...])
    mij = u2 * jax.nn.sigmoid(u2)
    mij_ref[...] = mij

    d2 = bond.shape[1] + mij.shape[1]
    colsumb = (jnp.sum(bw1a_ref[...], 0) + jnp.sum(bw1b_ref[...], 0))[None, :]
    s2 = jnp.sum(bond, 1, keepdims=True) + jnp.sum(mij, 1, keepdims=True)
    q2 = (jnp.sum(bond * bond, 1, keepdims=True)
          + jnp.sum(mij * mij, 1, keepdims=True))
    mean2 = s2 / d2
    inv2 = lax.rsqrt(q2 / d2 - mean2 * mean2 + 1e-5)
    ub = ((jnp.dot(bond.astype(bf16), bw1a_ref[...].astype(bf16),
                   preferred_element_type=f32)
           + jnp.dot(mij.astype(bf16), bw1b_ref[...].astype(bf16),
                     preferred_element_type=f32)
           - mean2 * colsumb) * inv2 + bb1_ref[...])
    t2 = ub * jax.nn.sigmoid(ub)
    m2 = jnp.mean(t2, -1, keepdims=True)
    c2 = t2 - m2
    t2n = c2 * lax.rsqrt(jnp.mean(c2 * c2, -1, keepdims=True) + 1e-5)
    ub2 = (jnp.dot(t2n.astype(bf16), bw2_ref[...].astype(bf16),
                   preferred_element_type=f32) + bb2_ref[...])
    bout_ref[...] = ub2 * jax.nn.sigmoid(ub2)


def _node_body(h_ref, a0_ref, a1_ref, nw1a_ref, nw1b_ref, nb1_ref,
               nw2_ref, nb2_ref, out_ref):
    f32 = jnp.float32
    bf16 = jnp.bfloat16
    h = h_ref[...]
    agg = a0_ref[...] + a1_ref[...]
    u = (jnp.dot(h.astype(bf16), nw1a_ref[...].astype(bf16),
                 preferred_element_type=f32)
         + jnp.dot(agg.astype(bf16), nw1b_ref[...].astype(bf16),
                   preferred_element_type=f32)
         + nb1_ref[...])
    t = u * jax.nn.sigmoid(u)
    m = jnp.mean(t, -1, keepdims=True)
    c = t - m
    tn = c * lax.rsqrt(jnp.mean(c * c, -1, keepdims=True) + 1e-5)
    out_ref[...] = (jnp.dot(tn.astype(bf16), nw2_ref[...].astype(bf16),
                            preferred_element_type=f32) + nb2_ref[...])


def _full(shape):
    nd = len(shape)
    return pl.BlockSpec(shape, lambda i, _nd=nd: (0,) * nd)


def kernel(x, h, edge_index, bond, e_w1, e_b1, e_w2, e_b2,
           b_w1, b_b1, b_w2, b_b2, n_w1, n_b1, n_w2, n_b2, offset, coeff):
    f32 = jnp.float32
    N, D = h.shape
    E = edge_index.shape[1]
    nrbf = offset.shape[0]
    db = bond.shape[1]
    hid2 = e_w1.shape[1]
    hid = e_w2.shape[1]

    row = edge_index[0].astype(jnp.int32)
    col = edge_index[1].astype(jnp.int32)
    hx = jnp.concatenate(
        [h, x.astype(f32), jnp.zeros((N, REC - D - 3), f32)], axis=1)

    hxs, hxd = _make_gather(N, E)(hx, row, col)

    EB = 2000
    grid = (E // EB,)
    w1a, w1b = e_w1[:D], e_w1[D:2 * D]
    w1c, w1d = e_w1[2 * D:2 * D + db], e_w1[2 * D + db:]
    bw1a, bw1b = b_w1[:db], b_w1[db:]
    in_specs = [
        pl.BlockSpec((EB, REC), lambda i: (i, 0)),
        pl.BlockSpec((EB, REC), lambda i: (i, 0)),
        pl.BlockSpec((EB, db), lambda i: (i, 0)),
        _full(w1a.shape), _full(w1b.shape), _full(w1c.shape), _full(w1d.shape),
        _full((1, hid2)), _full(e_w2.shape), _full((1, hid)),
        _full(bw1a.shape), _full(bw1b.shape), _full((1, hid)),
        _full(b_w2.shape), _full((1, db)),
        _full((1, nrbf)), _full((1, 1)),
    ]
    mij, bond_out = pl.pallas_call(
        _edge_body,
        grid=grid,
        in_specs=in_specs,
        out_specs=[pl.BlockSpec((EB, hid), lambda i: (i, 0)),
                   pl.BlockSpec((EB, db), lambda i: (i, 0))],
        out_shape=[jax.ShapeDtypeStruct((E, hid), f32),
                   jax.ShapeDtypeStruct((E, db), f32)],
    )(hxs, hxd, bond, w1a, w1b, w1c, w1d, e_b1.reshape(1, -1), e_w2,
      e_b2.reshape(1, -1), bw1a, bw1b, b_b1.reshape(1, -1), b_w2,
      b_b2.reshape(1, -1), offset.reshape(1, -1),
      jnp.reshape(coeff, (1, 1)).astype(f32))

    aggp = _make_scatter(N, E, hid)(mij, row)

    NB = 2000
    nw1a, nw1b = n_w1[:D], n_w1[D:]
    nblk = N // NB
    h_out = pl.pallas_call(
        _node_body,
        grid=(nblk,),
        in_specs=[
            pl.BlockSpec((NB, D), lambda i: (i, 0)),
            pl.BlockSpec((NB, hid), lambda i: (i, 0)),
            pl.BlockSpec((NB, hid), lambda i, _o=nblk: (i + _o, 0)),
            _full(nw1a.shape), _full(nw1b.shape), _full((1, hid)),
            _full(n_w2.shape), _full((1, D)),
        ],
        out_specs=pl.BlockSpec((NB, D), lambda i: (i, 0)),
        out_shape=jax.ShapeDtypeStruct((N, D), f32),
    )(h, aggp, aggp, nw1a, nw1b, n_b1.reshape(1, -1), n_w2,
      n_b2.reshape(1, -1))

    return (h_out, bond_out)


# speedup vs baseline: 1.3059x; 1.3059x over previous
import functools

import jax
import jax.numpy as jnp
from jax import lax
from jax.experimental import pallas as pl
from jax.experimental.pallas import tpu as pltpu
from jax.experimental.pallas import tpu_sc as plsc

NC, NS, L = 2, 16, 16
NW = NC * NS
CHUNK = 400
REC = 144

_SC_PARAMS = pltpu.CompilerParams(use_tc_tiling_on_sc=False)


def _sc_mesh():
    return plsc.VectorSubcoreMesh(
        core_axis_name="c", subcore_axis_name="s", num_cores=NC, num_subcores=NS
    )


def _make_gather_h(N, E, D):
    epw = E // NW
    n_chunks = epw // CHUNK

    @functools.partial(
        pl.kernel,
        out_type=[
            jax.ShapeDtypeStruct((E, D), jnp.float32),
            jax.ShapeDtypeStruct((E, D), jnp.float32),
        ],
        mesh=_sc_mesh(),
        scratch_types=[
            pltpu.VMEM((CHUNK,), jnp.int32),
            pltpu.VMEM((CHUNK,), jnp.int32),
            pltpu.VMEM((CHUNK, D), jnp.float32),
            pltpu.VMEM((CHUNK, D), jnp.float32),
            pltpu.SemaphoreType.DMA,
            pltpu.SemaphoreType.DMA,
        ],
    )
    def gather_k(h_hbm, row_hbm, col_hbm, hs_hbm, hd_hbm,
                 ir_v, ic_v, gs, gd, sem1, sem2):
        wid = lax.axis_index("s") * NC + lax.axis_index("c")
        base_w = wid * epw

        def chunk_body(ci, _):
            base = pl.multiple_of(base_w + ci * CHUNK, 8)
            pltpu.sync_copy(row_hbm.at[pl.ds(base, CHUNK)], ir_v)
            pltpu.sync_copy(col_hbm.at[pl.ds(base, CHUNK)], ic_v)
            cp1 = pltpu.make_async_copy(h_hbm.at[ir_v], gs, sem1)
            cp2 = pltpu.make_async_copy(h_hbm.at[ic_v], gd, sem2)
            cp1.start()
            cp2.start()
            cp1.wait()
            pltpu.sync_copy(gs, hs_hbm.at[pl.ds(base, CHUNK)])
            cp2.wait()
            pltpu.sync_copy(gd, hd_hbm.at[pl.ds(base, CHUNK)])
            return _

        lax.fori_loop(0, n_chunks, chunk_body, None)

    return gather_k


def _make_gather_x(N, E, DX):
    epw = E // NW
    n_chunks = epw // CHUNK

    @functools.partial(
        pl.kernel,
        out_type=[
            jax.ShapeDtypeStruct((E, DX), jnp.float32),
            jax.ShapeDtypeStruct((E, DX), jnp.float32),
        ],
        mesh=_sc_mesh(),
        compiler_params=_SC_PARAMS,
        scratch_types=[
            pltpu.VMEM((CHUNK,), jnp.int32),
            pltpu.VMEM((CHUNK,), jnp.int32),
            pltpu.VMEM((CHUNK, DX), jnp.float32),
            pltpu.VMEM((CHUNK, DX), jnp.float32),
            pltpu.SemaphoreType.DMA,
            pltpu.SemaphoreType.DMA,
        ],
    )
    def gather_k(xp_hbm, row_hbm, col_hbm, xr_hbm, xc_hbm,
                 ir_v, ic_v, gs, gd, sem1, sem2):
        wid = lax.axis_index("s") * NC + lax.axis_index("c")
        base_w = wid * epw

        def chunk_body(ci, _):
            base = pl.multiple_of(base_w + ci * CHUNK, 8)
            pltpu.sync_copy(row_hbm.at[pl.ds(base, CHUNK)], ir_v)
            pltpu.sync_copy(col_hbm.at[pl.ds(base, CHUNK)], ic_v)
            cp1 = pltpu.make_async_copy(xp_hbm.at[ir_v], gs, sem1)
            cp2 = pltpu.make_async_copy(xp_hbm.at[ic_v], gd, sem2)
            cp1.start()
            cp2.start()
            cp1.wait()
            pltpu.sync_copy(gs, xr_hbm.at[pl.ds(base, CHUNK)])
            cp2.wait()
            pltpu.sync_copy(gd, xc_hbm.at[pl.ds(base, CHUNK)])
            return _

        lax.fori_loop(0, n_chunks, chunk_body, None)

    return gather_k


SCHUNK = 200


def _make_scatter(N, E, D):
    epw = E // NW
    n_chunks = epw // SCHUNK
    npt = (N // NS) // 8 * 8
    tail = N - NS * npt

    @functools.partial(
        pl.kernel,
        out_type=jax.ShapeDtypeStruct((NC * N, D), jnp.float32),
        mesh=_sc_mesh(),
        scratch_types=[
            pltpu.VMEM((SCHUNK, D), jnp.float32),
            pltpu.VMEM((SCHUNK,), jnp.int32),
            pltpu.VMEM_SHARED((N, D), jnp.float32),
        ],
    )
    def scatter_k(mij_hbm, row_hbm, out_hbm, mbuf, idxv, agg_sh):
        cid = lax.axis_index("c")
        sid = lax.axis_index("s")
        wid = sid * NC + cid
        base_w = wid * epw

        def zrow(rr, _):
            def zcol(cc, _2):
                mbuf[rr, pl.ds(cc * L, L)] = jnp.zeros((L,), jnp.float32)
                return _2
            return lax.fori_loop(0, D // L, zcol, _)

        lax.fori_loop(0, SCHUNK, zrow, None)
        done = 0
        while done < npt:
            step = min(SCHUNK, npt - done)
            pltpu.sync_copy(mbuf.at[pl.ds(0, step)],
                            agg_sh.at[pl.ds(sid * npt + done, step)])
            done += step

        @pl.when(sid == NS - 1)
        def _():
            pltpu.sync_copy(mbuf.at[pl.ds(0, tail)],
                            agg_sh.at[pl.ds(NS * npt, tail)])

        plsc.subcore_barrier()

        def chunk_body(ci, _):
            base = pl.multiple_of(base_w + ci * SCHUNK, 8)
            pltpu.sync_copy(row_hbm.at[pl.ds(base, SCHUNK)], idxv)
            pltpu.sync_copy(mij_hbm.at[pl.ds(base, SCHUNK)], mbuf)
            pltpu.sync_copy(mbuf, agg_sh.at[idxv], add=True)
            return _

        lax.fori_loop(0, n_chunks, chunk_body, None)
        plsc.subcore_barrier()
        pltpu.sync_copy(agg_sh.at[pl.ds(sid * npt, npt)],
                        out_hbm.at[pl.ds(cid * N + sid * npt, npt)])

        @pl.when(sid == NS - 1)
        def _():
            pltpu.sync_copy(agg_sh.at[pl.ds(NS * npt, tail)],
                            out_hbm.at[pl.ds(cid * N + NS * npt, tail)])

    return scatter_k


def _edge_body(hs_ref, hd_ref, xr_ref, xc_ref, bond_ref,
               w1a_ref, w1b_ref, w1c_ref, w1d_ref, b1_ref, w2_ref, b2_ref,
               bw1a_ref, bw1b_ref, bb1_ref, bw2_ref, bb2_ref,
               off_ref, cf_ref, mij_ref, bout_ref):
    f32 = jnp.float32
    bf16 = jnp.bfloat16
    D = w1a_ref.shape[0]
    hs = hs_ref[...]
    hd = hd_ref[...]
    bond = bond_ref[...]
    dx = xr_ref[...] - xc_ref[...]
    r2 = jnp.sum(dx * dx, axis=1, keepdims=True)
    r = jnp.sqrt(r2 + 1e-8)
    rbf = jnp.exp(cf_ref[0, 0] * (r - off_ref[...]) ** 2)

    d1 = 2 * D + bond.shape[1] + rbf.shape[1]
    colsum1 = (jnp.sum(w1a_ref[...], 0) + jnp.sum(w1b_ref[...], 0)
               + jnp.sum(w1c_ref[...], 0) + jnp.sum(w1d_ref[...], 0))[None, :]
    s = (jnp.sum(hs, 1, keepdims=True) + jnp.sum(hd, 1, keepdims=True)
         + jnp.sum(bond, 1, keepdims=True) + jnp.sum(rbf, 1, keepdims=True))
    q = (jnp.sum(hs * hs, 1, keepdims=True) + jnp.sum(hd * hd, 1, keepdims=True)
         + jnp.sum(bond * bond, 1, keepdims=True)
         + jnp.sum(rbf * rbf, 1, keepdims=True))
    mean = s / d1
    inv = lax.rsqrt(q / d1 - mean * mean + 1e-5)
    msum = (jnp.dot(hs.astype(bf16), w1a_ref[...].astype(bf16),
                    preferred_element_type=f32)
            + jnp.dot(hd.astype(bf16), w1b_ref[...].astype(bf16),
                      preferred_element_type=f32)
            + jnp.dot(bond.astype(bf16), w1c_ref[...].astype(bf16),
                      preferred_element_type=f32)
            + jnp.dot(rbf.astype(bf16), w1d_ref[...].astype(bf16),
                      preferred_element_type=f32))
    u1 = (msum - mean * colsum1) * inv + b1_ref[...]
    t1 = u1 * jax.nn.sigmoid(u1)
    m1 = jnp.mean(t1, -1, keepdims=True)
    c1 = t1 - m1
    t1n = c1 * lax.rsqrt(jnp.mean(c1 * c1, -1, keepdims=True) + 1e-5)
    u2 = (jnp.dot(t1n.astype(bf16), w2_ref[...].astype(bf16),
                  preferred_element_type=f32) + b2_ref[...])
    mij = u2 * jax.nn.sigmoid(u2)
    mij_ref[...] = mij

    d2 = bond.shape[1] + mij.shape[1]
    colsumb = (jnp.sum(bw1a_ref[...], 0) + jnp.sum(bw1b_ref[...], 0))[None, :]
    s2 = jnp.sum(bond, 1, keepdims=True) + jnp.sum(mij, 1, keepdims=True)
    q2 = (jnp.sum(bond * bond, 1, keepdims=True)
          + jnp.sum(mij * mij, 1, keepdims=True))
    mean2 = s2 / d2
    inv2 = lax.rsqrt(q2 / d2 - mean2 * mean2 + 1e-5)
    ub = ((jnp.dot(bond.astype(bf16), bw1a_ref[...].astype(bf16),
                   preferred_element_type=f32)
           + jnp.dot(mij.astype(bf16), bw1b_ref[...].astype(bf16),
                     preferred_element_type=f32)
           - mean2 * colsumb) * inv2 + bb1_ref[...])
    t2 = ub * jax.nn.sigmoid(ub)
    m2 = jnp.mean(t2, -1, keepdims=True)
    c2 = t2 - m2
    t2n = c2 * lax.rsqrt(jnp.mean(c2 * c2, -1, keepdims=True) + 1e-5)
    ub2 = (jnp.dot(t2n.astype(bf16), bw2_ref[...].astype(bf16),
                   preferred_element_type=f32) + bb2_ref[...])
    bout_ref[...] = ub2 * jax.nn.sigmoid(ub2)


def _node_body(h_ref, a0_ref, a1_ref, nw1a_ref, nw1b_ref, nb1_ref,
               nw2_ref, nb2_ref, out_ref):
    f32 = jnp.float32
    bf16 = jnp.bfloat16
    h = h_ref[...]
    agg = a0_ref[...] + a1_ref[...]
    u = (jnp.dot(h.astype(bf16), nw1a_ref[...].astype(bf16),
                 preferred_element_type=f32)
         + jnp.dot(agg.astype(bf16), nw1b_ref[...].astype(bf16),
                   preferred_element_type=f32)
         + nb1_ref[...])
    t = u * jax.nn.sigmoid(u)
    m = jnp.mean(t, -1, keepdims=True)
    c = t - m
    tn = c * lax.rsqrt(jnp.mean(c * c, -1, keepdims=True) + 1e-5)
    out_ref[...] = (jnp.dot(tn.astype(bf16), nw2_ref[...].astype(bf16),
                            preferred_element_type=f32) + nb2_ref[...])


def _full(shape):
    nd = len(shape)
    return pl.BlockSpec(shape, lambda i, _nd=nd: (0,) * nd)


def kernel(x, h, edge_index, bond, e_w1, e_b1, e_w2, e_b2,
           b_w1, b_b1, b_w2, b_b2, n_w1, n_b1, n_w2, n_b2, offset, coeff):
    f32 = jnp.float32
    N, D = h.shape
    E = edge_index.shape[1]
    nrbf = offset.shape[0]
    db = bond.shape[1]
    hid2 = e_w1.shape[1]
    hid = e_w2.shape[1]

    row = edge_index[0].astype(jnp.int32)
    col = edge_index[1].astype(jnp.int32)
    DX = 16
    xp = jnp.concatenate([x.astype(f32), jnp.zeros((N, DX - 3), f32)], axis=1)

    hs, hd = _make_gather_h(N, E, D)(h, row, col)
    xr, xc = _make_gather_x(N, E, DX)(xp, row, col)

    EB = 2000
    grid = (E // EB,)
    w1a, w1b = e_w1[:D], e_w1[D:2 * D]
    w1c, w1d = e_w1[2 * D:2 * D + db], e_w1[2 * D + db:]
    bw1a, bw1b = b_w1[:db], b_w1[db:]
    in_specs = [
        pl.BlockSpec((EB, D), lambda i: (i, 0)),
        pl.BlockSpec((EB, D), lambda i: (i, 0)),
        pl.BlockSpec((EB, 16), lambda i: (i, 0)),
        pl.BlockSpec((EB, 16), lambda i: (i, 0)),
        pl.BlockSpec((EB, db), lambda i: (i, 0)),
        _full(w1a.shape), _full(w1b.shape), _full(w1c.shape), _full(w1d.shape),
        _full((1, hid2)), _full(e_w2.shape), _full((1, hid)),
        _full(bw1a.shape), _full(bw1b.shape), _full((1, hid)),
        _full(b_w2.shape), _full((1, db)),
        _full((1, nrbf)), _full((1, 1)),
    ]
    mij, bond_out = pl.pallas_call(
        _edge_body,
        grid=grid,
        in_specs=in_specs,
        out_specs=[pl.BlockSpec((EB, hid), lambda i: (i, 0)),
                   pl.BlockSpec((EB, db), lambda i: (i, 0))],
        out_shape=[jax.ShapeDtypeStruct((E, hid), f32),
                   jax.ShapeDtypeStruct((E, db), f32)],
    )(hs, hd, xr, xc, bond, w1a, w1b, w1c, w1d, e_b1.reshape(1, -1), e_w2,
      e_b2.reshape(1, -1), bw1a, bw1b, b_b1.reshape(1, -1), b_w2,
      b_b2.reshape(1, -1), offset.reshape(1, -1),
      jnp.reshape(coeff, (1, 1)).astype(f32))

    aggp = _make_scatter(N, E, hid)(mij, row)

    NB = 2000
    nw1a, nw1b = n_w1[:D], n_w1[D:]
    nblk = N // NB
    h_out = pl.pallas_call(
        _node_body,
        grid=(nblk,),
        in_specs=[
            pl.BlockSpec((NB, D), lambda i: (i, 0)),
            pl.BlockSpec((NB, hid), lambda i: (i, 0)),
            pl.BlockSpec((NB, hid), lambda i, _o=nblk: (i + _o, 0)),
            _full(nw1a.shape), _full(nw1b.shape), _full((1, hid)),
            _full(n_w2.shape), _full((1, D)),
        ],
        out_specs=pl.BlockSpec((NB, D), lambda i: (i, 0)),
        out_shape=jax.ShapeDtypeStruct((N, D), f32),
    )(h, aggp, aggp, nw1a, nw1b, n_b1.reshape(1, -1), n_w2,
      n_b2.reshape(1, -1))

    return (h_out, bond_out)


# speedup vs baseline: 1.3963x; 1.0693x over previous
import functools

import jax
import jax.numpy as jnp
from jax import lax
from jax.experimental import pallas as pl
from jax.experimental.pallas import tpu as pltpu
from jax.experimental.pallas import tpu_sc as plsc

NC, NS, L = 2, 16, 16
NW = NC * NS
CHUNK = 400
REC = 144

_SC_PARAMS = pltpu.CompilerParams(use_tc_tiling_on_sc=False)


def _sc_mesh():
    return plsc.VectorSubcoreMesh(
        core_axis_name="c", subcore_axis_name="s", num_cores=NC, num_subcores=NS
    )


def _make_gather_h(N, E, D):
    epw = E // NW
    n_chunks = epw // CHUNK

    @functools.partial(
        pl.kernel,
        out_type=[
            jax.ShapeDtypeStruct((E, D), jnp.float32),
            jax.ShapeDtypeStruct((E, D), jnp.float32),
        ],
        mesh=_sc_mesh(),
        scratch_types=[
            pltpu.VMEM((CHUNK,), jnp.int32),
            pltpu.VMEM((CHUNK,), jnp.int32),
            pltpu.VMEM((CHUNK, D), jnp.float32),
            pltpu.VMEM((CHUNK, D), jnp.float32),
            pltpu.SemaphoreType.DMA,
            pltpu.SemaphoreType.DMA,
        ],
    )
    def gather_k(h_hbm, row_hbm, col_hbm, hs_hbm, hd_hbm,
                 ir_v, ic_v, gs, gd, sem1, sem2):
        wid = lax.axis_index("s") * NC + lax.axis_index("c")
        base_w = wid * epw

        def chunk_body(ci, _):
            base = pl.multiple_of(base_w + ci * CHUNK, 8)
            pltpu.sync_copy(row_hbm.at[pl.ds(base, CHUNK)], ir_v)
            pltpu.sync_copy(col_hbm.at[pl.ds(base, CHUNK)], ic_v)
            cp1 = pltpu.make_async_copy(h_hbm.at[ir_v], gs, sem1)
            cp2 = pltpu.make_async_copy(h_hbm.at[ic_v], gd, sem2)
            cp1.start()
            cp2.start()
            cp1.wait()
            pltpu.sync_copy(gs, hs_hbm.at[pl.ds(base, CHUNK)])
            cp2.wait()
            pltpu.sync_copy(gd, hd_hbm.at[pl.ds(base, CHUNK)])
            return _

        lax.fori_loop(0, n_chunks, chunk_body, None)

    return gather_k


def _make_gather_x(N, E, DX):
    epw = E // NW
    n_chunks = epw // CHUNK

    @functools.partial(
        pl.kernel,
        out_type=[
            jax.ShapeDtypeStruct((E, DX), jnp.float32),
            jax.ShapeDtypeStruct((E, DX), jnp.float32),
        ],
        mesh=_sc_mesh(),
        compiler_params=_SC_PARAMS,
        scratch_types=[
            pltpu.VMEM((CHUNK,), jnp.int32),
            pltpu.VMEM((CHUNK,), jnp.int32),
            pltpu.VMEM((CHUNK, DX), jnp.float32),
            pltpu.VMEM((CHUNK, DX), jnp.float32),
            pltpu.SemaphoreType.DMA,
            pltpu.SemaphoreType.DMA,
        ],
    )
    def gather_k(xp_hbm, row_hbm, col_hbm, xr_hbm, xc_hbm,
                 ir_v, ic_v, gs, gd, sem1, sem2):
        wid = lax.axis_index("s") * NC + lax.axis_index("c")
        base_w = wid * epw

        def chunk_body(ci, _):
            base = pl.multiple_of(base_w + ci * CHUNK, 8)
            pltpu.sync_copy(row_hbm.at[pl.ds(base, CHUNK)], ir_v)
            pltpu.sync_copy(col_hbm.at[pl.ds(base, CHUNK)], ic_v)
            cp1 = pltpu.make_async_copy(xp_hbm.at[ir_v], gs, sem1)
            cp2 = pltpu.make_async_copy(xp_hbm.at[ic_v], gd, sem2)
            cp1.start()
            cp2.start()
            cp1.wait()
            pltpu.sync_copy(gs, xr_hbm.at[pl.ds(base, CHUNK)])
            cp2.wait()
            pltpu.sync_copy(gd, xc_hbm.at[pl.ds(base, CHUNK)])
            return _

        lax.fori_loop(0, n_chunks, chunk_body, None)

    return gather_k


SCHUNK = 200


def _make_scatter(N, E, D):
    epw = E // NW
    n_chunks = epw // SCHUNK
    npt = (N // NS) // 8 * 8
    tail = N - NS * npt

    @functools.partial(
        pl.kernel,
        out_type=jax.ShapeDtypeStruct((NC * N, D), jnp.float32),
        mesh=_sc_mesh(),
        scratch_types=[
            pltpu.VMEM((SCHUNK, D), jnp.float32),
            pltpu.VMEM((SCHUNK,), jnp.int32),
            pltpu.VMEM_SHARED((N, D), jnp.float32),
        ],
    )
    def scatter_k(mij_hbm, row_hbm, out_hbm, mbuf, idxv, agg_sh):
        cid = lax.axis_index("c")
        sid = lax.axis_index("s")
        wid = sid * NC + cid
        base_w = wid * epw

        def zrow(rr, _):
            def zcol(cc, _2):
                mbuf[rr, pl.ds(cc * L, L)] = jnp.zeros((L,), jnp.float32)
                return _2
            return lax.fori_loop(0, D // L, zcol, _)

        lax.fori_loop(0, SCHUNK, zrow, None)
        done = 0
        while done < npt:
            step = min(SCHUNK, npt - done)
            pltpu.sync_copy(mbuf.at[pl.ds(0, step)],
                            agg_sh.at[pl.ds(sid * npt + done, step)])
            done += step

        @pl.when(sid == NS - 1)
        def _():
            pltpu.sync_copy(mbuf.at[pl.ds(0, tail)],
                            agg_sh.at[pl.ds(NS * npt, tail)])

        plsc.subcore_barrier()

        def chunk_body(ci, _):
            base = pl.multiple_of(base_w + ci * SCHUNK, 8)
            pltpu.sync_copy(row_hbm.at[pl.ds(base, SCHUNK)], idxv)
            pltpu.sync_copy(mij_hbm.at[pl.ds(base, SCHUNK)], mbuf)
            pltpu.sync_copy(mbuf, agg_sh.at[idxv], add=True)
            return _

        lax.fori_loop(0, n_chunks, chunk_body, None)
        plsc.subcore_barrier()
        pltpu.sync_copy(agg_sh.at[pl.ds(sid * npt, npt)],
                        out_hbm.at[pl.ds(cid * N + sid * npt, npt)])

        @pl.when(sid == NS - 1)
        def _():
            pltpu.sync_copy(agg_sh.at[pl.ds(NS * npt, tail)],
                            out_hbm.at[pl.ds(cid * N + NS * npt, tail)])

    return scatter_k


def _prep_body(x_ref, h_ref, xp_ref):
    f32 = jnp.float32
    h = h_ref[...]
    nb = h.shape[0]
    ones = jnp.ones((h.shape[1], 1), f32)
    S = jnp.dot(h, ones, preferred_element_type=f32)
    Q = jnp.dot(h * h, ones, preferred_element_type=f32)
    xp_ref[...] = jnp.concatenate(
        [x_ref[...], jnp.zeros((nb, 5), f32), S, Q, jnp.zeros((nb, 6), f32)],
        axis=1)


def _edge_body(hs_ref, hd_ref, xr_ref, xc_ref, bond_ref,
               w1a_ref, w1b_ref, w1c_ref, w1d_ref, b1_ref, w2_ref, b2_ref,
               bw1a_ref, bw1b_ref, bb1_ref, bw2_ref, bb2_ref,
               off_ref, cf_ref, mij_ref, bout_ref):
    f32 = jnp.float32
    bf16 = jnp.bfloat16
    D = w1a_ref.shape[0]

    def dotf(a, b):
        return jnp.dot(a, b, preferred_element_type=f32)

    def dotb(a, b):
        return jnp.dot(a.astype(bf16), b, preferred_element_type=f32)

    def silu(v):
        t = 0.5 * v
        return t * jnp.tanh(t) + t

    hs = hs_ref[...]
    hd = hd_ref[...]
    bond = bond_ref[...]
    xr = xr_ref[...]
    xc = xc_ref[...]
    dx = xr[:, :8] - xc[:, :8]
    r2 = jnp.sum(dx * dx, 1, keepdims=True)
    xsum = xr + xc
    s_h = xsum[:, 8:9]
    q_h = xsum[:, 9:10]
    r = jnp.sqrt(r2 + 1e-8)
    rbf = jnp.exp(cf_ref[0, 0] * (r - off_ref[...]) ** 2)

    sB = jnp.sum(bond, 1, keepdims=True)
    qB = jnp.sum(bond * bond, 1, keepdims=True)
    d1 = 2 * D + bond.shape[1] + rbf.shape[1]
    s = s_h + sB + jnp.sum(rbf, 1, keepdims=True)
    q = q_h + qB + jnp.sum(rbf * rbf, 1, keepdims=True)
    mean = s * (1.0 / d1)
    inv = lax.rsqrt(q * (1.0 / d1) - mean * mean + 1e-5)
    colsum1 = (jnp.sum(w1a_ref[...].astype(f32), 0) + jnp.sum(w1b_ref[...].astype(f32), 0)
               + jnp.sum(w1c_ref[...].astype(f32), 0) + jnp.sum(w1d_ref[...].astype(f32), 0))[None, :]
    msum = (dotb(hs, w1a_ref[...]) + dotb(hd, w1b_ref[...])
            + dotb(bond, w1c_ref[...]) + dotb(rbf, w1d_ref[...]))
    u1 = (msum - mean * colsum1) * inv + b1_ref[...]
    t1 = silu(u1)

    hid2 = t1.shape[1]
    m1 = jnp.sum(t1, 1, keepdims=True) * (1.0 / hid2)
    q1 = jnp.sum(t1 * t1, 1, keepdims=True) * (1.0 / hid2)
    inv1 = lax.rsqrt(q1 - m1 * m1 + 1e-5)
    colsum2 = jnp.sum(w2_ref[...].astype(f32), 0)[None, :]
    u2 = (dotb(t1, w2_ref[...]) - m1 * colsum2) * inv1 + b2_ref[...]
    mij = silu(u2)
    mij_ref[...] = mij

    hid = mij.shape[1]
    d2 = bond.shape[1] + hid
    s2 = sB + jnp.sum(mij, 1, keepdims=True)
    q2 = qB + jnp.sum(mij * mij, 1, keepdims=True)
    mean2 = s2 * (1.0 / d2)
    inv2 = lax.rsqrt(q2 * (1.0 / d2) - mean2 * mean2 + 1e-5)
    colsumb = (jnp.sum(bw1a_ref[...].astype(f32), 0) + jnp.sum(bw1b_ref[...].astype(f32), 0))[None, :]
    ub = (dotb(bond, bw1a_ref[...]) + dotb(mij, bw1b_ref[...])
          - mean2 * colsumb) * inv2 + bb1_ref[...]
    t2 = silu(ub)
    m2 = jnp.sum(t2, 1, keepdims=True) * (1.0 / hid)
    q2b = jnp.sum(t2 * t2, 1, keepdims=True) * (1.0 / hid)
    inv2b = lax.rsqrt(q2b - m2 * m2 + 1e-5)
    colsumb2 = jnp.sum(bw2_ref[...].astype(f32), 0)[None, :]
    ub2 = (dotb(t2, bw2_ref[...]) - m2 * colsumb2) * inv2b + bb2_ref[...]
    bout_ref[...] = silu(ub2)


def _node_body(h_ref, a0_ref, a1_ref, nw1a_ref, nw1b_ref, nb1_ref,
               nw2_ref, nb2_ref, out_ref):
    f32 = jnp.float32
    bf16 = jnp.bfloat16
    h = h_ref[...]
    agg = a0_ref[...] + a1_ref[...]
    u = (jnp.dot(h.astype(bf16), nw1a_ref[...], preferred_element_type=f32)
         + jnp.dot(agg.astype(bf16), nw1b_ref[...], preferred_element_type=f32)
         + nb1_ref[...])
    t = u * jax.nn.sigmoid(u)
    m = jnp.mean(t, -1, keepdims=True)
    c = t - m
    tn = c * lax.rsqrt(jnp.mean(c * c, -1, keepdims=True) + 1e-5)
    out_ref[...] = (jnp.dot(tn.astype(bf16), nw2_ref[...],
                            preferred_element_type=f32) + nb2_ref[...])


def _full(shape):
    nd = len(shape)
    return pl.BlockSpec(shape, lambda i, _nd=nd: (0,) * nd)


def kernel(x, h, edge_index, bond, e_w1, e_b1, e_w2, e_b2,
           b_w1, b_b1, b_w2, b_b2, n_w1, n_b1, n_w2, n_b2, offset, coeff):
    f32 = jnp.float32
    N, D = h.shape
    E = edge_index.shape[1]
    nrbf = offset.shape[0]
    db = bond.shape[1]
    hid2 = e_w1.shape[1]
    hid = e_w2.shape[1]

    row = edge_index[0].astype(jnp.int32)
    col = edge_index[1].astype(jnp.int32)
    DX = 16
    NBP = 2000
    xp = pl.pallas_call(
        _prep_body,
        grid=(N // NBP,),
        in_specs=[pl.BlockSpec((NBP, 3), lambda i: (i, 0)),
                  pl.BlockSpec((NBP, D), lambda i: (i, 0))],
        out_specs=pl.BlockSpec((NBP, DX), lambda i: (i, 0)),
        out_shape=jax.ShapeDtypeStruct((N, DX), f32),
    )(x.astype(f32), h)

    hs, hd = _make_gather_h(N, E, D)(h, row, col)
    xr, xc = _make_gather_x(N, E, DX)(xp, row, col)

    EB = 2000
    grid = (E // EB,)
    bf16 = jnp.bfloat16
    w1a, w1b = e_w1[:D].astype(bf16), e_w1[D:2 * D].astype(bf16)
    w1c = e_w1[2 * D:2 * D + db].astype(bf16)
    w1d = e_w1[2 * D + db:].astype(bf16)
    e_w2b = e_w2.astype(bf16)
    bw1a, bw1b = b_w1[:db].astype(bf16), b_w1[db:].astype(bf16)
    b_w2b = b_w2.astype(bf16)
    in_specs = [
        pl.BlockSpec((EB, D), lambda i: (i, 0)),
        pl.BlockSpec((EB, D), lambda i: (i, 0)),
        pl.BlockSpec((EB, 16), lambda i: (i, 0)),
        pl.BlockSpec((EB, 16), lambda i: (i, 0)),
        pl.BlockSpec((EB, db), lambda i: (i, 0)),
        _full(w1a.shape), _full(w1b.shape), _full(w1c.shape), _full(w1d.shape),
        _full((1, hid2)), _full(e_w2b.shape), _full((1, hid)),
        _full(bw1a.shape), _full(bw1b.shape), _full((1, hid)),
        _full(b_w2b.shape), _full((1, db)),
        _full((1, nrbf)), _full((1, 1)),
    ]
    mij, bond_out = pl.pallas_call(
        _edge_body,
        grid=grid,
        in_specs=in_specs,
        out_specs=[pl.BlockSpec((EB, hid), lambda i: (i, 0)),
                   pl.BlockSpec((EB, db), lambda i: (i, 0))],
        out_shape=[jax.ShapeDtypeStruct((E, hid), f32),
                   jax.ShapeDtypeStruct((E, db), f32)],
    )(hs, hd, xr, xc, bond, w1a, w1b, w1c, w1d, e_b1.reshape(1, -1), e_w2b,
      e_b2.reshape(1, -1), bw1a, bw1b, b_b1.reshape(1, -1), b_w2b,
      b_b2.reshape(1, -1), offset.reshape(1, -1),
      jnp.reshape(coeff, (1, 1)).astype(f32))

    aggp = _make_scatter(N, E, hid)(mij, row)

    NB = 2000
    nw1a, nw1b = n_w1[:D].astype(bf16), n_w1[D:].astype(bf16)
    n_w2b = n_w2.astype(bf16)
    nblk = N // NB
    h_out = pl.pallas_call(
        _node_body,
        grid=(nblk,),
        in_specs=[
            pl.BlockSpec((NB, D), lambda i: (i, 0)),
            pl.BlockSpec((NB, hid), lambda i: (i, 0)),
            pl.BlockSpec((NB, hid), lambda i, _o=nblk: (i + _o, 0)),
            _full(nw1a.shape), _full(nw1b.shape), _full((1, hid)),
            _full(n_w2b.shape), _full((1, D)),
        ],
        out_specs=pl.BlockSpec((NB, D), lambda i: (i, 0)),
        out_shape=jax.ShapeDtypeStruct((N, D), f32),
    )(h, aggp, aggp, nw1a, nw1b, n_b1.reshape(1, -1), n_w2b,
      n_b2.reshape(1, -1))

    return (h_out, bond_out)


# speedup vs baseline: 1.4239x; 1.0198x over previous
import functools

import jax
import jax.numpy as jnp
from jax import lax
from jax.experimental import pallas as pl
from jax.experimental.pallas import tpu as pltpu
from jax.experimental.pallas import tpu_sc as plsc

NC, NS, L = 2, 16, 16
NW = NC * NS
CHUNK = 200
REC = 144

_SC_PARAMS = pltpu.CompilerParams(use_tc_tiling_on_sc=False)


def _sc_mesh():
    return plsc.VectorSubcoreMesh(
        core_axis_name="c", subcore_axis_name="s", num_cores=NC, num_subcores=NS
    )


def _make_gather_h(N, E, D):
    epw = E // NW
    n_chunks = epw // CHUNK

    @functools.partial(
        pl.kernel,
        out_type=[
            jax.ShapeDtypeStruct((E, D), jnp.float32),
            jax.ShapeDtypeStruct((E, D), jnp.float32),
        ],
        mesh=_sc_mesh(),
        scratch_types=[
            pltpu.VMEM((CHUNK,), jnp.int32),
            pltpu.VMEM((CHUNK,), jnp.int32),
            pltpu.VMEM((CHUNK, D), jnp.float32),
            pltpu.VMEM((CHUNK, D), jnp.float32),
            pltpu.SemaphoreType.DMA,
            pltpu.SemaphoreType.DMA,
        ],
    )
    def gather_k(h_hbm, row_hbm, col_hbm, hs_hbm, hd_hbm,
                 ir_v, ic_v, gs, gd, sem1, sem2):
        wid = lax.axis_index("s") * NC + lax.axis_index("c")
        base_w = wid * epw

        def chunk_body(ci, _):
            base = pl.multiple_of(base_w + ci * CHUNK, 8)
            pltpu.sync_copy(row_hbm.at[pl.ds(base, CHUNK)], ir_v)
            pltpu.sync_copy(col_hbm.at[pl.ds(base, CHUNK)], ic_v)
            cp1 = pltpu.make_async_copy(h_hbm.at[ir_v], gs, sem1)
            cp2 = pltpu.make_async_copy(h_hbm.at[ic_v], gd, sem2)
            cp1.start()
            cp2.start()
            cp1.wait()
            pltpu.sync_copy(gs, hs_hbm.at[pl.ds(base, CHUNK)])
            cp2.wait()
            pltpu.sync_copy(gd, hd_hbm.at[pl.ds(base, CHUNK)])
            return _

        lax.fori_loop(0, n_chunks, chunk_body, None)

    return gather_k


def _make_gather_x(N, E, DX):
    epw = E // NW
    n_chunks = epw // CHUNK

    @functools.partial(
        pl.kernel,
        out_type=[
            jax.ShapeDtypeStruct((E, DX), jnp.float32),
            jax.ShapeDtypeStruct((E, DX), jnp.float32),
        ],
        mesh=_sc_mesh(),
        compiler_params=_SC_PARAMS,
        scratch_types=[
            pltpu.VMEM((CHUNK,), jnp.int32),
            pltpu.VMEM((CHUNK,), jnp.int32),
            pltpu.VMEM((CHUNK, DX), jnp.float32),
            pltpu.VMEM((CHUNK, DX), jnp.float32),
            pltpu.SemaphoreType.DMA,
            pltpu.SemaphoreType.DMA,
        ],
    )
    def gather_k(xp_hbm, row_hbm, col_hbm, xr_hbm, xc_hbm,
                 ir_v, ic_v, gs, gd, sem1, sem2):
        wid = lax.axis_index("s") * NC + lax.axis_index("c")
        base_w = wid * epw

        def chunk_body(ci, _):
            base = pl.multiple_of(base_w + ci * CHUNK, 8)
            pltpu.sync_copy(row_hbm.at[pl.ds(base, CHUNK)], ir_v)
            pltpu.sync_copy(col_hbm.at[pl.ds(base, CHUNK)], ic_v)
            cp1 = pltpu.make_async_copy(xp_hbm.at[ir_v], gs, sem1)
            cp2 = pltpu.make_async_copy(xp_hbm.at[ic_v], gd, sem2)
            cp1.start()
            cp2.start()
            cp1.wait()
            pltpu.sync_copy(gs, xr_hbm.at[pl.ds(base, CHUNK)])
            cp2.wait()
            pltpu.sync_copy(gd, xc_hbm.at[pl.ds(base, CHUNK)])
            return _

        lax.fori_loop(0, n_chunks, chunk_body, None)

    return gather_k


SCHUNK = 200


def _make_scatter(N, E, D):
    epw = E // NW
    n_chunks = epw // SCHUNK
    npt = (N // NS) // 8 * 8
    tail = N - NS * npt

    @functools.partial(
        pl.kernel,
        out_type=jax.ShapeDtypeStruct((NC * N, D), jnp.float32),
        mesh=_sc_mesh(),
        scratch_types=[
            pltpu.VMEM((SCHUNK, D), jnp.float32),
            pltpu.VMEM((SCHUNK,), jnp.int32),
            pltpu.VMEM_SHARED((N, D), jnp.float32),
        ],
    )
    def scatter_k(mij_hbm, row_hbm, out_hbm, mbuf, idxv, agg_sh):
        cid = lax.axis_index("c")
        sid = lax.axis_index("s")
        wid = sid * NC + cid
        base_w = wid * epw

        def zrow(rr, _):
            def zcol(cc, _2):
                mbuf[rr, pl.ds(cc * L, L)] = jnp.zeros((L,), jnp.float32)
                return _2
            return lax.fori_loop(0, D // L, zcol, _)

        lax.fori_loop(0, SCHUNK, zrow, None)
        done = 0
        while done < npt:
            step = min(SCHUNK, npt - done)
            pltpu.sync_copy(mbuf.at[pl.ds(0, step)],
                            agg_sh.at[pl.ds(sid * npt + done, step)])
            done += step

        @pl.when(sid == NS - 1)
        def _():
            pltpu.sync_copy(mbuf.at[pl.ds(0, tail)],
                            agg_sh.at[pl.ds(NS * npt, tail)])

        plsc.subcore_barrier()

        def chunk_body(ci, _):
            base = pl.multiple_of(base_w + ci * SCHUNK, 8)
            pltpu.sync_copy(row_hbm.at[pl.ds(base, SCHUNK)], idxv)
            pltpu.sync_copy(mij_hbm.at[pl.ds(base, SCHUNK)], mbuf)
            pltpu.sync_copy(mbuf, agg_sh.at[idxv], add=True)
            return _

        lax.fori_loop(0, n_chunks, chunk_body, None)
        plsc.subcore_barrier()
        pltpu.sync_copy(agg_sh.at[pl.ds(sid * npt, npt)],
                        out_hbm.at[pl.ds(cid * N + sid * npt, npt)])

        @pl.when(sid == NS - 1)
        def _():
            pltpu.sync_copy(agg_sh.at[pl.ds(NS * npt, tail)],
                            out_hbm.at[pl.ds(cid * N + NS * npt, tail)])

    return scatter_k


def _prep_body(x_ref, h_ref, xp_ref):
    f32 = jnp.float32
    h = h_ref[...]
    nb = h.shape[0]
    ones = jnp.ones((h.shape[1], 1), f32)
    S = jnp.dot(h, ones, preferred_element_type=f32)
    Q = jnp.dot(h * h, ones, preferred_element_type=f32)
    xp_ref[...] = jnp.concatenate(
        [x_ref[...], jnp.zeros((nb, 5), f32), S, Q, jnp.zeros((nb, 6), f32)],
        axis=1)


def _edge_body(hs_ref, hd_ref, xr_ref, xc_ref, bond_ref,
               w1a_ref, w1b_ref, w1c_ref, w1d_ref, b1_ref, w2_ref, b2_ref,
               bw1a_ref, bw1b_ref, bb1_ref, bw2_ref, bb2_ref,
               off_ref, cf_ref, mij_ref, bout_ref):
    f32 = jnp.float32
    bf16 = jnp.bfloat16
    D = w1a_ref.shape[0]

    def dotf(a, b):
        return jnp.dot(a, b, preferred_element_type=f32)

    def dotb(a, b):
        return jnp.dot(a.astype(bf16), b, preferred_element_type=f32)

    def silu(v):
        t = 0.5 * v
        return t * jnp.tanh(t) + t

    hs = hs_ref[...]
    hd = hd_ref[...]
    bond = bond_ref[...]
    xr = xr_ref[...]
    xc = xc_ref[...]
    dx = xr[:, :8] - xc[:, :8]
    r2 = jnp.sum(dx * dx, 1, keepdims=True)
    xsum = xr + xc
    s_h = xsum[:, 8:9]
    q_h = xsum[:, 9:10]
    r = jnp.sqrt(r2 + 1e-8)
    rbf = jnp.exp(cf_ref[0, 0] * (r - off_ref[...]) ** 2)

    sB = jnp.sum(bond, 1, keepdims=True)
    qB = jnp.sum(bond * bond, 1, keepdims=True)
    d1 = 2 * D + bond.shape[1] + rbf.shape[1]
    s = s_h + sB + jnp.sum(rbf, 1, keepdims=True)
    q = q_h + qB + jnp.sum(rbf * rbf, 1, keepdims=True)
    mean = s * (1.0 / d1)
    inv = lax.rsqrt(q * (1.0 / d1) - mean * mean + 1e-5)
    colsum1 = (jnp.sum(w1a_ref[...].astype(f32), 0) + jnp.sum(w1b_ref[...].astype(f32), 0)
               + jnp.sum(w1c_ref[...].astype(f32), 0) + jnp.sum(w1d_ref[...].astype(f32), 0))[None, :]
    msum = (dotb(hs, w1a_ref[...]) + dotb(hd, w1b_ref[...])
            + dotb(bond, w1c_ref[...]) + dotb(rbf, w1d_ref[...]))
    u1 = (msum - mean * colsum1) * inv + b1_ref[...]
    t1 = silu(u1)

    hid2 = t1.shape[1]
    m1 = jnp.sum(t1, 1, keepdims=True) * (1.0 / hid2)
    q1 = jnp.sum(t1 * t1, 1, keepdims=True) * (1.0 / hid2)
    inv1 = lax.rsqrt(q1 - m1 * m1 + 1e-5)
    colsum2 = jnp.sum(w2_ref[...].astype(f32), 0)[None, :]
    u2 = (dotb(t1, w2_ref[...]) - m1 * colsum2) * inv1 + b2_ref[...]
    mij = silu(u2)
    mij_ref[...] = mij

    hid = mij.shape[1]
    d2 = bond.shape[1] + hid
    s2 = sB + jnp.sum(mij, 1, keepdims=True)
    q2 = qB + jnp.sum(mij * mij, 1, keepdims=True)
    mean2 = s2 * (1.0 / d2)
    inv2 = lax.rsqrt(q2 * (1.0 / d2) - mean2 * mean2 + 1e-5)
    colsumb = (jnp.sum(bw1a_ref[...].astype(f32), 0) + jnp.sum(bw1b_ref[...].astype(f32), 0))[None, :]
    ub = (dotb(bond, bw1a_ref[...]) + dotb(mij, bw1b_ref[...])
          - mean2 * colsumb) * inv2 + bb1_ref[...]
    t2 = silu(ub)
    m2 = jnp.sum(t2, 1, keepdims=True) * (1.0 / hid)
    q2b = jnp.sum(t2 * t2, 1, keepdims=True) * (1.0 / hid)
    inv2b = lax.rsqrt(q2b - m2 * m2 + 1e-5)
    colsumb2 = jnp.sum(bw2_ref[...].astype(f32), 0)[None, :]
    ub2 = (dotb(t2, bw2_ref[...]) - m2 * colsumb2) * inv2b + bb2_ref[...]
    bout_ref[...] = silu(ub2)


def _node_body(h_ref, a0_ref, a1_ref, a2_ref, a3_ref,
               nw1a_ref, nw1b_ref, nb1_ref, nw2_ref, nb2_ref, out_ref):
    f32 = jnp.float32
    bf16 = jnp.bfloat16
    h = h_ref[...]
    agg = (a0_ref[...] + a1_ref[...]) + (a2_ref[...] + a3_ref[...])
    u = (jnp.dot(h.astype(bf16), nw1a_ref[...], preferred_element_type=f32)
         + jnp.dot(agg.astype(bf16), nw1b_ref[...], preferred_element_type=f32)
         + nb1_ref[...])
    t = u * jax.nn.sigmoid(u)
    m = jnp.mean(t, -1, keepdims=True)
    c = t - m
    tn = c * lax.rsqrt(jnp.mean(c * c, -1, keepdims=True) + 1e-5)
    out_ref[...] = (jnp.dot(tn.astype(bf16), nw2_ref[...],
                            preferred_element_type=f32) + nb2_ref[...])


def _full(shape):
    nd = len(shape)
    return pl.BlockSpec(shape, lambda i, _nd=nd: (0,) * nd)


def kernel(x, h, edge_index, bond, e_w1, e_b1, e_w2, e_b2,
           b_w1, b_b1, b_w2, b_b2, n_w1, n_b1, n_w2, n_b2, offset, coeff):
    f32 = jnp.float32
    N, D = h.shape
    E = edge_index.shape[1]
    nrbf = offset.shape[0]
    db = bond.shape[1]
    hid2 = e_w1.shape[1]
    hid = e_w2.shape[1]

    row = edge_index[0].astype(jnp.int32)
    col = edge_index[1].astype(jnp.int32)
    DX = 16
    NBP = 2000
    xp = pl.pallas_call(
        _prep_body,
        grid=(N // NBP,),
        in_specs=[pl.BlockSpec((NBP, 3), lambda i: (i, 0)),
                  pl.BlockSpec((NBP, D), lambda i: (i, 0))],
        out_specs=pl.BlockSpec((NBP, DX), lambda i: (i, 0)),
        out_shape=jax.ShapeDtypeStruct((N, DX), f32),
    )(x.astype(f32), h)

    xr, xc = _make_gather_x(N, E, DX)(xp, row, col)

    EH = E // 2
    gat = _make_gather_h(N, EH, D)
    sct = _make_scatter(N, EH, hid)
    EB = 2000
    bf16 = jnp.bfloat16
    w1a, w1b = e_w1[:D].astype(bf16), e_w1[D:2 * D].astype(bf16)
    w1c = e_w1[2 * D:2 * D + db].astype(bf16)
    w1d = e_w1[2 * D + db:].astype(bf16)
    e_w2b = e_w2.astype(bf16)
    bw1a, bw1b = b_w1[:db].astype(bf16), b_w1[db:].astype(bf16)
    b_w2b = b_w2.astype(bf16)

    def edge_half(half, hs, hd):
        off = half * (EH // EB)
        in_specs = [
            pl.BlockSpec((EB, D), lambda i: (i, 0)),
            pl.BlockSpec((EB, D), lambda i: (i, 0)),
            pl.BlockSpec((EB, DX), lambda i, _o=off: (i + _o, 0)),
            pl.BlockSpec((EB, DX), lambda i, _o=off: (i + _o, 0)),
            pl.BlockSpec((EB, db), lambda i, _o=off: (i + _o, 0)),
            _full(w1a.shape), _full(w1b.shape), _full(w1c.shape),
            _full(w1d.shape),
            _full((1, hid2)), _full(e_w2b.shape), _full((1, hid)),
            _full(bw1a.shape), _full(bw1b.shape), _full((1, hid)),
            _full(b_w2b.shape), _full((1, db)),
            _full((1, nrbf)), _full((1, 1)),
        ]
        return pl.pallas_call(
            _edge_body,
            grid=(EH // EB,),
            in_specs=in_specs,
            out_specs=[pl.BlockSpec((EB, hid), lambda i: (i, 0)),
                       pl.BlockSpec((EB, db), lambda i: (i, 0))],
            out_shape=[jax.ShapeDtypeStruct((EH, hid), f32),
                       jax.ShapeDtypeStruct((EH, db), f32)],
        )(hs, hd, xr, xc, bond, w1a, w1b, w1c, w1d, e_b1.reshape(1, -1),
          e_w2b, e_b2.reshape(1, -1), bw1a, bw1b, b_b1.reshape(1, -1),
          b_w2b, b_b2.reshape(1, -1), offset.reshape(1, -1),
          jnp.reshape(coeff, (1, 1)).astype(f32))

    rows = [row[:EH], row[EH:]]
    cols = [col[:EH], col[EH:]]
    hs0, hd0 = gat(h, rows[0], cols[0])
    hs1, hd1 = gat(h, rows[1], cols[1])
    mij0, bout0 = edge_half(0, hs0, hd0)
    aggp0 = sct(mij0, rows[0])
    mij1, bout1 = edge_half(1, hs1, hd1)
    aggp1 = sct(mij1, rows[1])
    bond_out = jnp.concatenate([bout0, bout1], axis=0)

    NB = 2000
    nw1a, nw1b = n_w1[:D].astype(bf16), n_w1[D:].astype(bf16)
    n_w2b = n_w2.astype(bf16)
    nblk = N // NB
    h_out = pl.pallas_call(
        _node_body,
        grid=(nblk,),
        in_specs=[
            pl.BlockSpec((NB, D), lambda i: (i, 0)),
            pl.BlockSpec((NB, hid), lambda i: (i, 0)),
            pl.BlockSpec((NB, hid), lambda i, _o=nblk: (i + _o, 0)),
            pl.BlockSpec((NB, hid), lambda i: (i, 0)),
            pl.BlockSpec((NB, hid), lambda i, _o=nblk: (i + _o, 0)),
            _full(nw1a.shape), _full(nw1b.shape), _full((1, hid)),
            _full(n_w2b.shape), _full((1, D)),
        ],
        out_specs=pl.BlockSpec((NB, D), lambda i: (i, 0)),
        out_shape=jax.ShapeDtypeStruct((N, D), f32),
    )(h, aggp0, aggp0, aggp1, aggp1, nw1a, nw1b, n_b1.reshape(1, -1),
      n_w2b, n_b2.reshape(1, -1))

    return (h_out, bond_out)


# speedup vs baseline: 1.5035x; 1.0559x over previous
import functools

import jax
import jax.numpy as jnp
from jax import lax
from jax.experimental import pallas as pl
from jax.experimental.pallas import tpu as pltpu
from jax.experimental.pallas import tpu_sc as plsc

NC, NS, L = 2, 16, 16
NW = NC * NS
CHUNK = 200
REC = 144

_SC_PARAMS = pltpu.CompilerParams(use_tc_tiling_on_sc=False)


def _sc_mesh():
    return plsc.VectorSubcoreMesh(
        core_axis_name="c", subcore_axis_name="s", num_cores=NC, num_subcores=NS
    )


def _make_gather_h(N, E, D):
    epw = E // NW
    n_chunks = epw // CHUNK

    @functools.partial(
        pl.kernel,
        out_type=[
            jax.ShapeDtypeStruct((E, D), jnp.float32),
            jax.ShapeDtypeStruct((E, D), jnp.float32),
        ],
        mesh=_sc_mesh(),
        scratch_types=[
            pltpu.VMEM((CHUNK,), jnp.int32),
            pltpu.VMEM((CHUNK,), jnp.int32),
            pltpu.VMEM((CHUNK, D), jnp.float32),
            pltpu.VMEM((CHUNK, D), jnp.float32),
            pltpu.SemaphoreType.DMA,
            pltpu.SemaphoreType.DMA,
        ],
    )
    def gather_k(h_hbm, row_hbm, col_hbm, hs_hbm, hd_hbm,
                 ir_v, ic_v, gs, gd, sem1, sem2):
        wid = lax.axis_index("s") * NC + lax.axis_index("c")
        base_w = wid * epw

        def chunk_body(ci, _):
            base = pl.multiple_of(base_w + ci * CHUNK, 8)
            pltpu.sync_copy(row_hbm.at[pl.ds(base, CHUNK)], ir_v)
            pltpu.sync_copy(col_hbm.at[pl.ds(base, CHUNK)], ic_v)
            cp1 = pltpu.make_async_copy(h_hbm.at[ir_v], gs, sem1)
            cp2 = pltpu.make_async_copy(h_hbm.at[ic_v], gd, sem2)
            cp1.start()
            cp2.start()
            cp1.wait()
            pltpu.sync_copy(gs, hs_hbm.at[pl.ds(base, CHUNK)])
            cp2.wait()
            pltpu.sync_copy(gd, hd_hbm.at[pl.ds(base, CHUNK)])
            return _

        lax.fori_loop(0, n_chunks, chunk_body, None)

    return gather_k


def _make_gather_x(N, E, DX):
    epw = E // NW
    n_chunks = epw // CHUNK

    @functools.partial(
        pl.kernel,
        out_type=jax.ShapeDtypeStruct((E, DX), jnp.float32),
        mesh=_sc_mesh(),
        compiler_params=_SC_PARAMS,
        scratch_types=[
            pltpu.VMEM((CHUNK,), jnp.int32),
            pltpu.VMEM((CHUNK, DX), jnp.float32),
            pltpu.SemaphoreType.DMA,
        ],
    )
    def gather_k(xp_hbm, idx_hbm, out_hbm, iv, gb, sem):
        wid = lax.axis_index("s") * NC + lax.axis_index("c")
        base_w = wid * epw

        def chunk_body(ci, _):
            base = pl.multiple_of(base_w + ci * CHUNK, 8)
            pltpu.sync_copy(idx_hbm.at[pl.ds(base, CHUNK)], iv)
            pltpu.async_copy(xp_hbm.at[iv], gb, sem).wait()
            pltpu.sync_copy(gb, out_hbm.at[pl.ds(base, CHUNK)])
            return _

        lax.fori_loop(0, n_chunks, chunk_body, None)

    return gather_k


SCHUNK = 200


def _make_scatter(N, E, D):
    epw = E // NW
    n_chunks = epw // SCHUNK
    npt = (N // NS) // 8 * 8
    tail = N - NS * npt

    @functools.partial(
        pl.kernel,
        out_type=jax.ShapeDtypeStruct((NC * N, D), jnp.float32),
        mesh=_sc_mesh(),
        scratch_types=[
            pltpu.VMEM((SCHUNK, D), jnp.float32),
            pltpu.VMEM((SCHUNK,), jnp.int32),
            pltpu.VMEM_SHARED((N, D), jnp.float32),
        ],
    )
    def scatter_k(mij_hbm, row_hbm, out_hbm, mbuf, idxv, agg_sh):
        cid = lax.axis_index("c")
        sid = lax.axis_index("s")
        wid = sid * NC + cid
        base_w = wid * epw

        def zrow(rr, _):
            def zcol(cc, _2):
                mbuf[rr, pl.ds(cc * L, L)] = jnp.zeros((L,), jnp.float32)
                return _2
            return lax.fori_loop(0, D // L, zcol, _)

        lax.fori_loop(0, SCHUNK, zrow, None)
        done = 0
        while done < npt:
            step = min(SCHUNK, npt - done)
            pltpu.sync_copy(mbuf.at[pl.ds(0, step)],
                            agg_sh.at[pl.ds(sid * npt + done, step)])
            done += step

        @pl.when(sid == NS - 1)
        def _():
            pltpu.sync_copy(mbuf.at[pl.ds(0, tail)],
                            agg_sh.at[pl.ds(NS * npt, tail)])

        plsc.subcore_barrier()

        def chunk_body(ci, _):
            base = pl.multiple_of(base_w + ci * SCHUNK, 8)
            pltpu.sync_copy(row_hbm.at[pl.ds(base, SCHUNK)], idxv)
            pltpu.sync_copy(mij_hbm.at[pl.ds(base, SCHUNK)], mbuf)
            pltpu.sync_copy(mbuf, agg_sh.at[idxv], add=True)
            return _

        lax.fori_loop(0, n_chunks, chunk_body, None)
        plsc.subcore_barrier()
        pltpu.sync_copy(agg_sh.at[pl.ds(sid * npt, npt)],
                        out_hbm.at[pl.ds(cid * N + sid * npt, npt)])

        @pl.when(sid == NS - 1)
        def _():
            pltpu.sync_copy(agg_sh.at[pl.ds(NS * npt, tail)],
                            out_hbm.at[pl.ds(cid * N + NS * npt, tail)])

    return scatter_k


def _prep_body(x_ref, h_ref, xp_ref):
    f32 = jnp.float32
    h = h_ref[...]
    nb = h.shape[0]
    ones = jnp.ones((h.shape[1], 1), f32)
    S = jnp.dot(h, ones, preferred_element_type=f32)
    Q = jnp.dot(h * h, ones, preferred_element_type=f32)
    xp_ref[...] = jnp.concatenate(
        [x_ref[...], jnp.zeros((nb, 5), f32), S, Q, jnp.zeros((nb, 6), f32)],
        axis=1)


def _edge_body(hs_ref, hd_ref, xr_ref, xc_ref, bond_ref,
               w1a_ref, w1b_ref, w1c_ref, w1d_ref, b1_ref, w2_ref, b2_ref,
               bw1a_ref, bw1b_ref, bb1_ref, bw2_ref, bb2_ref,
               off_ref, cf_ref, mij_ref, bout_ref):
    f32 = jnp.float32
    bf16 = jnp.bfloat16
    D = w1a_ref.shape[0]

    def dotf(a, b):
        return jnp.dot(a, b, preferred_element_type=f32)

    def dotb(a, b):
        return jnp.dot(a.astype(bf16), b, preferred_element_type=f32)

    def silu(v):
        t = 0.5 * v
        return t * jnp.tanh(t) + t

    hs = hs_ref[...]
    hd = hd_ref[...]
    bond = bond_ref[...]
    xr = xr_ref[...]
    xc = xc_ref[...]
    dx = xr[:, :8] - xc[:, :8]
    r2 = jnp.sum(dx * dx, 1, keepdims=True)
    xsum = xr + xc
    s_h = xsum[:, 8:9]
    q_h = xsum[:, 9:10]
    r = jnp.sqrt(r2 + 1e-8)
    rbf = jnp.exp(cf_ref[0, 0] * (r - off_ref[...]) ** 2)

    sB = jnp.sum(bond, 1, keepdims=True)
    qB = jnp.sum(bond * bond, 1, keepdims=True)
    d1 = 2 * D + bond.shape[1] + rbf.shape[1]
    s = s_h + sB + jnp.sum(rbf, 1, keepdims=True)
    q = q_h + qB + jnp.sum(rbf * rbf, 1, keepdims=True)
    mean = s * (1.0 / d1)
    inv = lax.rsqrt(q * (1.0 / d1) - mean * mean + 1e-5)
    colsum1 = (jnp.sum(w1a_ref[...].astype(f32), 0) + jnp.sum(w1b_ref[...].astype(f32), 0)
               + jnp.sum(w1c_ref[...].astype(f32), 0) + jnp.sum(w1d_ref[...].astype(f32), 0))[None, :]
    msum = (dotb(hs, w1a_ref[...]) + dotb(hd, w1b_ref[...])
            + dotb(bond, w1c_ref[...]) + dotb(rbf, w1d_ref[...]))
    u1 = (msum - mean * colsum1) * inv + b1_ref[...]
    t1 = silu(u1)

    hid2 = t1.shape[1]
    m1 = jnp.sum(t1, 1, keepdims=True) * (1.0 / hid2)
    q1 = jnp.sum(t1 * t1, 1, keepdims=True) * (1.0 / hid2)
    inv1 = lax.rsqrt(q1 - m1 * m1 + 1e-5)
    colsum2 = jnp.sum(w2_ref[...].astype(f32), 0)[None, :]
    u2 = (dotb(t1, w2_ref[...]) - m1 * colsum2) * inv1 + b2_ref[...]
    mij = silu(u2)
    mij_ref[...] = mij

    hid = mij.shape[1]
    d2 = bond.shape[1] + hid
    s2 = sB + jnp.sum(mij, 1, keepdims=True)
    q2 = qB + jnp.sum(mij * mij, 1, keepdims=True)
    mean2 = s2 * (1.0 / d2)
    inv2 = lax.rsqrt(q2 * (1.0 / d2) - mean2 * mean2 + 1e-5)
    colsumb = (jnp.sum(bw1a_ref[...].astype(f32), 0) + jnp.sum(bw1b_ref[...].astype(f32), 0))[None, :]
    ub = (dotb(bond, bw1a_ref[...]) + dotb(mij, bw1b_ref[...])
          - mean2 * colsumb) * inv2 + bb1_ref[...]
    t2 = silu(ub)
    m2 = jnp.sum(t2, 1, keepdims=True) * (1.0 / hid)
    q2b = jnp.sum(t2 * t2, 1, keepdims=True) * (1.0 / hid)
    inv2b = lax.rsqrt(q2b - m2 * m2 + 1e-5)
    colsumb2 = jnp.sum(bw2_ref[...].astype(f32), 0)[None, :]
    ub2 = (dotb(t2, bw2_ref[...]) - m2 * colsumb2) * inv2b + bb2_ref[...]
    bout_ref[...] = silu(ub2)


def _node_body(h_ref, a0_ref, a1_ref, a2_ref, a3_ref,
               nw1a_ref, nw1b_ref, nb1_ref, nw2_ref, nb2_ref, out_ref):
    f32 = jnp.float32
    bf16 = jnp.bfloat16
    h = h_ref[...]
    agg = (a0_ref[...] + a1_ref[...]) + (a2_ref[...] + a3_ref[...])
    u = (jnp.dot(h.astype(bf16), nw1a_ref[...], preferred_element_type=f32)
         + jnp.dot(agg.astype(bf16), nw1b_ref[...], preferred_element_type=f32)
         + nb1_ref[...])
    t = u * jax.nn.sigmoid(u)
    m = jnp.mean(t, -1, keepdims=True)
    c = t - m
    tn = c * lax.rsqrt(jnp.mean(c * c, -1, keepdims=True) + 1e-5)
    out_ref[...] = (jnp.dot(tn.astype(bf16), nw2_ref[...],
                            preferred_element_type=f32) + nb2_ref[...])


def _full(shape):
    nd = len(shape)
    return pl.BlockSpec(shape, lambda i, _nd=nd: (0,) * nd)


def kernel(x, h, edge_index, bond, e_w1, e_b1, e_w2, e_b2,
           b_w1, b_b1, b_w2, b_b2, n_w1, n_b1, n_w2, n_b2, offset, coeff):
    f32 = jnp.float32
    N, D = h.shape
    E = edge_index.shape[1]
    nrbf = offset.shape[0]
    db = bond.shape[1]
    hid2 = e_w1.shape[1]
    hid = e_w2.shape[1]

    row = edge_index[0].astype(jnp.int32)
    col = edge_index[1].astype(jnp.int32)
    DX = 16
    NBP = 2000
    xp = pl.pallas_call(
        _prep_body,
        grid=(N // NBP,),
        in_specs=[pl.BlockSpec((NBP, 3), lambda i: (i, 0)),
                  pl.BlockSpec((NBP, D), lambda i: (i, 0))],
        out_specs=pl.BlockSpec((NBP, DX), lambda i: (i, 0)),
        out_shape=jax.ShapeDtypeStruct((N, DX), f32),
    )(x.astype(f32), h)

    gx = _make_gather_x(N, E, DX)
    xc = gx(xp, col)
    xr = gx(xp, row)

    EH = E // 2
    gat = _make_gather_h(N, EH, D)
    sct = _make_scatter(N, EH, hid)
    EB = 2000
    bf16 = jnp.bfloat16
    w1a, w1b = e_w1[:D].astype(bf16), e_w1[D:2 * D].astype(bf16)
    w1c = e_w1[2 * D:2 * D + db].astype(bf16)
    w1d = e_w1[2 * D + db:].astype(bf16)
    e_w2b = e_w2.astype(bf16)
    bw1a, bw1b = b_w1[:db].astype(bf16), b_w1[db:].astype(bf16)
    b_w2b = b_w2.astype(bf16)

    def edge_half(half, hs, hd):
        off = half * (EH // EB)
        in_specs = [
            pl.BlockSpec((EB, D), lambda i: (i, 0)),
            pl.BlockSpec((EB, D), lambda i: (i, 0)),
            pl.BlockSpec((EB, DX), lambda i, _o=off: (i + _o, 0)),
            pl.BlockSpec((EB, DX), lambda i, _o=off: (i + _o, 0)),
            pl.BlockSpec((EB, db), lambda i, _o=off: (i + _o, 0)),
            _full(w1a.shape), _full(w1b.shape), _full(w1c.shape),
            _full(w1d.shape),
            _full((1, hid2)), _full(e_w2b.shape), _full((1, hid)),
            _full(bw1a.shape), _full(bw1b.shape), _full((1, hid)),
            _full(b_w2b.shape), _full((1, db)),
            _full((1, nrbf)), _full((1, 1)),
        ]
        return pl.pallas_call(
            _edge_body,
            grid=(EH // EB,),
            in_specs=in_specs,
            out_specs=[pl.BlockSpec((EB, hid), lambda i: (i, 0)),
                       pl.BlockSpec((EB, db), lambda i: (i, 0))],
            out_shape=[jax.ShapeDtypeStruct((EH, hid), f32),
                       jax.ShapeDtypeStruct((EH, db), f32)],
        )(hs, hd, xr, xc, bond, w1a, w1b, w1c, w1d, e_b1.reshape(1, -1),
          e_w2b, e_b2.reshape(1, -1), bw1a, bw1b, b_b1.reshape(1, -1),
          b_w2b, b_b2.reshape(1, -1), offset.reshape(1, -1),
          jnp.reshape(coeff, (1, 1)).astype(f32))

    rows = [row[:EH], row[EH:]]
    cols = [col[:EH], col[EH:]]
    hs0, hd0 = gat(h, rows[0], cols[0])
    hs1, hd1 = gat(h, rows[1], cols[1])
    mij0, bout0 = edge_half(0, hs0, hd0)
    aggp0 = sct(mij0, rows[0])
    mij1, bout1 = edge_half(1, hs1, hd1)
    aggp1 = sct(mij1, rows[1])
    bond_out = jnp.concatenate([bout0, bout1], axis=0)

    NB = 2000
    nw1a, nw1b = n_w1[:D].astype(bf16), n_w1[D:].astype(bf16)
    n_w2b = n_w2.astype(bf16)
    nblk = N // NB
    h_out = pl.pallas_call(
        _node_body,
        grid=(nblk,),
        in_specs=[
            pl.BlockSpec((NB, D), lambda i: (i, 0)),
            pl.BlockSpec((NB, hid), lambda i: (i, 0)),
            pl.BlockSpec((NB, hid), lambda i, _o=nblk: (i + _o, 0)),
            pl.BlockSpec((NB, hid), lambda i: (i, 0)),
            pl.BlockSpec((NB, hid), lambda i, _o=nblk: (i + _o, 0)),
            _full(nw1a.shape), _full(nw1b.shape), _full((1, hid)),
            _full(n_w2b.shape), _full((1, D)),
        ],
        out_specs=pl.BlockSpec((NB, D), lambda i: (i, 0)),
        out_shape=jax.ShapeDtypeStruct((N, D), f32),
    )(h, aggp0, aggp0, aggp1, aggp1, nw1a, nw1b, n_b1.reshape(1, -1),
      n_w2b, n_b2.reshape(1, -1))

    return (h_out, bond_out)
